# baseline (speedup 1.0000x reference)
import functools
import jax
import jax.numpy as jnp
from jax import lax
from jax.experimental import pallas as pl
from jax.experimental.pallas import tpu as pltpu
from jax.experimental.pallas import tpu_sc as plsc

B = 128
CLS = 8
POOL = 1024
HALF = 6
D = 512
SEQ = 77
HEAD = 1 + 2 * HALF
SUF = SEQ - HEAD
BPS = 4
ROWS = BPS * CLS

_NC = 2
_NS = 16
_NW = _NC * _NS
_RPW = POOL // _NW


def _tc_body(idx_ref, *refs):
    gathers = refs[:4 * BPS]
    (pref, suf, tokr, nctokr, out_p, out_tok, out_nctok) = refs[4 * BPS:]

    s = pl.program_id(0)
    for m in range(BPS):
        b = s * BPS + m
        g0, g1 = gathers[2 * m], gathers[2 * m + 1]
        a0, a1 = gathers[2 * BPS + 2 * m], gathers[2 * BPS + 2 * m + 1]
        is_g = b < (B // 2)
        r0v = jnp.where(is_g, g0[0], a0[0])
        r1v = jnp.where(is_g, g1[0], a1[0])
        ctx = jnp.concatenate([r0v, r1v], axis=0)
        lo = m * CLS
        out_p[lo:lo + CLS, 0:1, :] = pref[:]
        out_p[lo:lo + CLS, 1:HEAD, :] = jnp.broadcast_to(ctx[None],
                                                         (CLS, 2 * HALF, D))
        out_p[lo:lo + CLS, HEAD:SEQ, :] = suf[:]
        out_tok[lo:lo + CLS, :] = tokr[:]
        out_nctok[lo:lo + CLS, :] = jnp.broadcast_to(nctokr[:], (CLS, SEQ))


_G0 = 1
_A0 = 1 + POOL * HALF
_S0 = 1 + 2 * POOL * HALF
_HGATH = 16


_NSLOT = 3


def _nc_body(tbl, tmpl_init, out, tmpl, osem0, osem1, osem2, gsem0, gsem1,
             gsem2):
    osems = (osem0, osem1, osem2)
    gsems = (gsem0, gsem1, gsem2)
    c = lax.axis_index("c")
    s = lax.axis_index("s")
    wid = s * _NC + c
    base = wid * _RPW
    for t in range(_NSLOT):
        pltpu.sync_copy(tmpl_init.at[0], tmpl.at[t])

    lane = lax.iota(jnp.int32, _HGATH)
    base_vec = jnp.where(
        lane == 0, 0,
        jnp.where(lane <= HALF, _G0 + (lane - 1),
                  jnp.where(lane <= 2 * HALF, _A0 + (lane - 1 - HALF),
                            _S0 + (lane - HEAD))))
    scale_vec = jnp.where((lane >= 1) & (lane <= 2 * HALF), HALF, 0)

    def gather(i):
        t = i % _NSLOT
        idx = base_vec + (base + i) * scale_vec
        pltpu.make_async_copy(tbl.at[idx], tmpl.at[t, pl.ds(0, _HGATH)],
                              gsems[t]).start()

    def wait_gather(i):
        t = i % _NSLOT
        idx = base_vec + (base + i) * scale_vec
        pltpu.make_async_copy(tbl.at[idx], tmpl.at[t, pl.ds(0, _HGATH)],
                              gsems[t]).wait()

    def start_out(i):
        t = i % _NSLOT
        pltpu.make_async_copy(tmpl.at[t], out.at[base + i], osems[t]).start()

    def wait_out(i):
        t = i % _NSLOT
        pltpu.make_async_copy(tmpl.at[t], out.at[base + i], osems[t]).wait()

    for i in range(_RPW):
        if i >= _NSLOT:
            wait_out(i - _NSLOT)
        gather(i)
        if i >= 1:
            wait_gather(i - 1)
            start_out(i - 1)
    wait_gather(_RPW - 1)
    start_out(_RPW - 1)
    for i in range(_RPW - _NSLOT, _RPW):
        wait_out(i)


def kernel(indices_g, global_prompt, attribute_prompt, token_prefix,
           token_suffix, nc_token_prefix, nc_token_suffix,
           tokenized_prompts, nc_tokenized_prompts):
    tbl = jnp.concatenate([
        nc_token_prefix.reshape(1, D),
        global_prompt.reshape(POOL * HALF, D),
        attribute_prompt.reshape(POOL * HALF, D),
        nc_token_suffix.reshape(SUF, D),
    ], axis=0)
    tmpl_init = jnp.concatenate([
        nc_token_prefix,
        jnp.zeros((1, 2 * HALF, D), jnp.float32),
        nc_token_suffix,
    ], axis=1)
    nc_prompts = pl.kernel(
        _nc_body,
        out_type=jax.ShapeDtypeStruct((POOL, SEQ, D), jnp.float32),
        mesh=plsc.VectorSubcoreMesh(core_axis_name="c", subcore_axis_name="s",
                                    num_cores=_NC, num_subcores=_NS),
        scratch_types=[
            pltpu.VMEM((_NSLOT, SEQ, D), jnp.float32),
            pltpu.SemaphoreType.DMA,
            pltpu.SemaphoreType.DMA,
            pltpu.SemaphoreType.DMA,
            pltpu.SemaphoreType.DMA,
            pltpu.SemaphoreType.DMA,
            pltpu.SemaphoreType.DMA,
        ],
    )(tbl, tmpl_init)

    grid = (B // BPS,)

    def gspec(m):
        return pl.BlockSpec(
            (1, HALF, D),
            lambda s, idx, m=m: (idx[(2 * BPS * s + m) % B], 0, 0))

    in_specs = (
        [gspec(m) for m in range(2 * BPS)] +
        [gspec(m) for m in range(2 * BPS)] +
        [
            pl.BlockSpec((CLS, 1, D), lambda s, idx: (0, 0, 0)),
            pl.BlockSpec((CLS, SUF, D), lambda s, idx: (0, 0, 0)),
            pl.BlockSpec((CLS, SEQ), lambda s, idx: (0, 0)),
            pl.BlockSpec((1, SEQ), lambda s, idx: (0, 0)),
        ])
    out_specs = [
        pl.BlockSpec((ROWS, SEQ, D), lambda s, idx: (s, 0, 0)),
        pl.BlockSpec((ROWS, SEQ), lambda s, idx: (s, 0)),
        pl.BlockSpec((ROWS, SEQ), lambda s, idx: (s, 0)),
    ]
    out_shape = [
        jax.ShapeDtypeStruct((B * CLS, SEQ, D), jnp.float32),
        jax.ShapeDtypeStruct((B * CLS, SEQ), jnp.int32),
        jax.ShapeDtypeStruct((POOL, SEQ), jnp.int32),
    ]

    grid_spec = pltpu.PrefetchScalarGridSpec(
        num_scalar_prefetch=1,
        grid=grid,
        in_specs=in_specs,
        out_specs=out_specs,
    )
    prompts, tok, nc_tok = pl.pallas_call(
        _tc_body,
        grid_spec=grid_spec,
        out_shape=out_shape,
    )(indices_g,
      *([global_prompt] * (2 * BPS)), *([attribute_prompt] * (2 * BPS)),
      token_prefix, token_suffix, tokenized_prompts, nc_tokenized_prompts)

    return (prompts, tok, nc_prompts, nc_tok)


# speedup vs baseline: 1.0487x; 1.0487x over previous
import functools
import jax
import jax.numpy as jnp
from jax import lax
from jax.experimental import pallas as pl
from jax.experimental.pallas import tpu as pltpu
from jax.experimental.pallas import tpu_sc as plsc

B = 128
CLS = 8
POOL = 1024
HALF = 6
D = 512
SEQ = 77
HEAD = 1 + 2 * HALF
SUF = SEQ - HEAD
BPS = 4
ROWS = BPS * CLS

_NC = 2
_NS = 16
_NW = _NC * _NS
_RPW = POOL // _NW


def _tc_body(idx_ref, *refs):
    gathers = refs[:4 * BPS]
    (pref, suf, tokr, nctokr, out_p, out_tok, out_nctok) = refs[4 * BPS:]

    s = pl.program_id(0)
    for m in range(BPS):
        b = s * BPS + m
        g0, g1 = gathers[2 * m], gathers[2 * m + 1]
        a0, a1 = gathers[2 * BPS + 2 * m], gathers[2 * BPS + 2 * m + 1]
        is_g = b < (B // 2)
        r0v = jnp.where(is_g, g0[0], a0[0])
        r1v = jnp.where(is_g, g1[0], a1[0])
        ctx = jnp.concatenate([r0v, r1v], axis=0)
        lo = m * CLS
        out_p[lo:lo + CLS, 0:1, :] = pref[:]
        out_p[lo:lo + CLS, 1:HEAD, :] = jnp.broadcast_to(ctx[None],
                                                         (CLS, 2 * HALF, D))
        out_p[lo:lo + CLS, HEAD:SEQ, :] = suf[:]
        out_tok[lo:lo + CLS, :] = tokr[:]
        out_nctok[lo:lo + CLS, :] = jnp.broadcast_to(nctokr[:], (CLS, SEQ))


_G0 = 1
_A0 = 1 + POOL * HALF
_S0 = 1 + 2 * POOL * HALF
_HGATH = 16


def _nc_body(tbl, tmpl_init, out, stmpl, hbuf, osem, gsem0, gsem1):
    gsems = (gsem0, gsem1)
    c = lax.axis_index("c")
    s = lax.axis_index("s")
    wid = s * _NC + c
    base = wid * _RPW
    pltpu.sync_copy(tmpl_init.at[0], stmpl.at[s])

    lane = lax.iota(jnp.int32, _HGATH)
    base_vec = jnp.where(
        lane == 0, 0,
        jnp.where(lane <= HALF, _G0 + (lane - 1),
                  jnp.where(lane <= 2 * HALF, _A0 + (lane - 1 - HALF),
                            _S0 + (lane - HEAD))))
    scale_vec = jnp.where((lane >= 1) & (lane <= 2 * HALF), HALF, 0)

    def gather(i):
        t = i % 2
        idx = base_vec + (base + i) * scale_vec
        pltpu.make_async_copy(tbl.at[idx], hbuf.at[t], gsems[t]).start()

    def wait_gather(i):
        t = i % 2
        idx = base_vec + (base + i) * scale_vec
        pltpu.make_async_copy(tbl.at[idx], hbuf.at[t], gsems[t]).wait()

    def start_out(i):
        pltpu.make_async_copy(stmpl.at[s], out.at[base + i], osem).start()

    def wait_out(i):
        pltpu.make_async_copy(stmpl.at[s], out.at[base + i], osem).wait()

    gather(0)
    for i in range(_RPW):
        wait_gather(i)
        if i + 1 < _RPW:
            gather(i + 1)
        if i >= 1:
            wait_out(i - 1)
        pltpu.sync_copy(hbuf.at[i % 2], stmpl.at[s, pl.ds(0, _HGATH)])
        start_out(i)
    wait_out(_RPW - 1)


def kernel(indices_g, global_prompt, attribute_prompt, token_prefix,
           token_suffix, nc_token_prefix, nc_token_suffix,
           tokenized_prompts, nc_tokenized_prompts):
    tbl = jnp.concatenate([
        nc_token_prefix.reshape(1, D),
        global_prompt.reshape(POOL * HALF, D),
        attribute_prompt.reshape(POOL * HALF, D),
        nc_token_suffix.reshape(SUF, D),
    ], axis=0)
    tmpl_init = jnp.concatenate([
        nc_token_prefix,
        jnp.zeros((1, 2 * HALF, D), jnp.float32),
        nc_token_suffix,
    ], axis=1)
    nc_prompts = pl.kernel(
        _nc_body,
        out_type=jax.ShapeDtypeStruct((POOL, SEQ, D), jnp.float32),
        mesh=plsc.VectorSubcoreMesh(core_axis_name="c", subcore_axis_name="s",
                                    num_cores=_NC, num_subcores=_NS),
        scratch_types=[
            pltpu.VMEM_SHARED((_NS, SEQ, D), jnp.float32),
            pltpu.VMEM((2, _HGATH, D), jnp.float32),
            pltpu.SemaphoreType.DMA,
            pltpu.SemaphoreType.DMA,
            pltpu.SemaphoreType.DMA,
        ],
    )(tbl, tmpl_init)

    grid = (B // BPS,)

    def gspec(m):
        return pl.BlockSpec(
            (1, HALF, D),
            lambda s, idx, m=m: (idx[(2 * BPS * s + m) % B], 0, 0))

    in_specs = (
        [gspec(m) for m in range(2 * BPS)] +
        [gspec(m) for m in range(2 * BPS)] +
        [
            pl.BlockSpec((CLS, 1, D), lambda s, idx: (0, 0, 0)),
            pl.BlockSpec((CLS, SUF, D), lambda s, idx: (0, 0, 0)),
            pl.BlockSpec((CLS, SEQ), lambda s, idx: (0, 0)),
            pl.BlockSpec((1, SEQ), lambda s, idx: (0, 0)),
        ])
    out_specs = [
        pl.BlockSpec((ROWS, SEQ, D), lambda s, idx: (s, 0, 0)),
        pl.BlockSpec((ROWS, SEQ), lambda s, idx: (s, 0)),
        pl.BlockSpec((ROWS, SEQ), lambda s, idx: (s, 0)),
    ]
    out_shape = [
        jax.ShapeDtypeStruct((B * CLS, SEQ, D), jnp.float32),
        jax.ShapeDtypeStruct((B * CLS, SEQ), jnp.int32),
        jax.ShapeDtypeStruct((POOL, SEQ), jnp.int32),
    ]

    grid_spec = pltpu.PrefetchScalarGridSpec(
        num_scalar_prefetch=1,
        grid=grid,
        in_specs=in_specs,
        out_specs=out_specs,
    )
    prompts, tok, nc_tok = pl.pallas_call(
        _tc_body,
        grid_spec=grid_spec,
        out_shape=out_shape,
    )(indices_g,
      *([global_prompt] * (2 * BPS)), *([attribute_prompt] * (2 * BPS)),
      token_prefix, token_suffix, tokenized_prompts, nc_tokenized_prompts)

    return (prompts, tok, nc_prompts, nc_tok)


# speedup vs baseline: 1.6218x; 1.5466x over previous
import jax
import jax.numpy as jnp
from jax.experimental import pallas as pl
from jax.experimental.pallas import tpu as pltpu

B = 128
CLS = 8
POOL = 1024
HALF = 6
D = 512
SEQ = 77
HEAD = 1 + 2 * HALF
SUF = SEQ - HEAD
BPS = 4
ROWS = BPS * CLS


def _body(idx_ref, *refs):
    gathers = refs[:4 * BPS]
    (pref, suf, ncpref, ncsuf, gid, aid, tokr, nctokr,
     out_p, out_tok, out_ncp, out_nctok) = refs[4 * BPS:]

    s = pl.program_id(0)
    for m in range(BPS):
        b = s * BPS + m
        g0, g1 = gathers[2 * m], gathers[2 * m + 1]
        a0, a1 = gathers[2 * BPS + 2 * m], gathers[2 * BPS + 2 * m + 1]
        is_g = b < (B // 2)
        r0v = jnp.where(is_g, g0[0], a0[0])
        r1v = jnp.where(is_g, g1[0], a1[0])
        ctx = jnp.concatenate([r0v, r1v], axis=0)
        lo = m * CLS
        out_p[lo:lo + CLS, 0:1, :] = pref[:]
        out_p[lo:lo + CLS, 1:HEAD, :] = jnp.broadcast_to(ctx[None],
                                                         (CLS, 2 * HALF, D))
        out_p[lo:lo + CLS, HEAD:SEQ, :] = suf[:]
        out_tok[lo:lo + CLS, :] = tokr[:]
        out_nctok[lo:lo + CLS, :] = jnp.broadcast_to(nctokr[:], (CLS, SEQ))
    out_ncp[:, 0:1, :] = jnp.broadcast_to(ncpref[:], (ROWS, 1, D))
    out_ncp[:, 1:1 + HALF, :] = gid[:]
    out_ncp[:, 1 + HALF:HEAD, :] = aid[:]
    out_ncp[:, HEAD:SEQ, :] = jnp.broadcast_to(ncsuf[:], (ROWS, SUF, D))


def kernel(indices_g, global_prompt, attribute_prompt, token_prefix,
           token_suffix, nc_token_prefix, nc_token_suffix,
           tokenized_prompts, nc_tokenized_prompts):
    grid = (B // BPS,)

    def gspec(m):
        return pl.BlockSpec(
            (1, HALF, D),
            lambda s, idx, m=m: (idx[(2 * BPS * s + m) % B], 0, 0))

    in_specs = (
        [gspec(m) for m in range(2 * BPS)] +
        [gspec(m) for m in range(2 * BPS)] +
        [
            pl.BlockSpec((CLS, 1, D), lambda s, idx: (0, 0, 0)),
            pl.BlockSpec((CLS, SUF, D), lambda s, idx: (0, 0, 0)),
            pl.BlockSpec((1, 1, D), lambda s, idx: (0, 0, 0)),
            pl.BlockSpec((1, SUF, D), lambda s, idx: (0, 0, 0)),
            pl.BlockSpec((ROWS, HALF, D), lambda s, idx: (s, 0, 0)),
            pl.BlockSpec((ROWS, HALF, D), lambda s, idx: (s, 0, 0)),
            pl.BlockSpec((CLS, SEQ), lambda s, idx: (0, 0)),
            pl.BlockSpec((1, SEQ), lambda s, idx: (0, 0)),
        ])
    out_specs = [
        pl.BlockSpec((ROWS, SEQ, D), lambda s, idx: (s, 0, 0)),
        pl.BlockSpec((ROWS, SEQ), lambda s, idx: (s, 0)),
        pl.BlockSpec((ROWS, SEQ, D), lambda s, idx: (s, 0, 0)),
        pl.BlockSpec((ROWS, SEQ), lambda s, idx: (s, 0)),
    ]
    out_shape = [
        jax.ShapeDtypeStruct((B * CLS, SEQ, D), jnp.float32),
        jax.ShapeDtypeStruct((B * CLS, SEQ), jnp.int32),
        jax.ShapeDtypeStruct((POOL, SEQ, D), jnp.float32),
        jax.ShapeDtypeStruct((POOL, SEQ), jnp.int32),
    ]

    grid_spec = pltpu.PrefetchScalarGridSpec(
        num_scalar_prefetch=1,
        grid=grid,
        in_specs=in_specs,
        out_specs=out_specs,
    )
    prompts, tok, nc_prompts, nc_tok = pl.pallas_call(
        _body,
        grid_spec=grid_spec,
        out_shape=out_shape,
    )(indices_g,
      *([global_prompt] * (2 * BPS)), *([attribute_prompt] * (2 * BPS)),
      token_prefix, token_suffix, nc_token_prefix, nc_token_suffix,
      global_prompt, attribute_prompt, tokenized_prompts,
      nc_tokenized_prompts)

    return (prompts, tok, nc_prompts, nc_tok)
